# baseline (speedup 1.0000x reference)
import functools

import jax
import jax.numpy as jnp
from jax.experimental import pallas as pl
from jax.experimental.pallas import tpu as pltpu

_BQ = 256
_BK = 1024


def _patchcore_kernel(f2_ref, m_ref, qsq_ref, msq_ref,
                      min_ref, idx_ref, score_ref, *, n_chunks, k_total):
    f2 = f2_ref[...]
    q_sq = qsq_ref[...]

    def body(ki, carry):
        best_d, best_i = carry
        m = m_ref[pl.ds(ki * _BK, _BK), :]
        m_sq = msq_ref[:, pl.ds(ki * _BK, _BK)]
        cross2 = jax.lax.dot_general(
            f2, m, (((1,), (1,)), ((), ())),
            preferred_element_type=jnp.float32)
        dist = jnp.sqrt(jnp.maximum((q_sq + m_sq) + cross2, 0.0))
        bminv = jnp.min(dist, axis=1)
        bidx = jnp.argmin(dist, axis=1).astype(jnp.int32)
        bidx = bidx + ki * _BK
        take = bminv < best_d
        return (jnp.where(take, bminv, best_d),
                jnp.where(take, bidx, best_i))

    init = (jnp.full((_BQ,), jnp.inf, dtype=jnp.float32),
            jnp.zeros((_BQ,), dtype=jnp.int32))
    best_d, best_i = jax.lax.fori_loop(0, n_chunks, body, init)
    min_ref[...] = best_d
    idx_ref[...] = best_i

    block_max = jnp.max(best_d)[None, None]
    qi = pl.program_id(0)

    @pl.when(qi == 0)
    def _():
        score_ref[...] = block_max

    @pl.when(qi != 0)
    def _():
        score_ref[...] = jnp.maximum(score_ref[...], block_max)


def kernel(features, patch_memory):
    q, d = features.shape
    k, _ = patch_memory.shape
    n_chunks = k // _BK

    q_sq = jnp.sum(features * features, axis=1, keepdims=True)
    m_sq = jnp.sum(patch_memory * patch_memory, axis=1)[None, :]
    f2 = features * -2.0

    body = functools.partial(_patchcore_kernel, n_chunks=n_chunks, k_total=k)

    min_d, nn_idx, score = pl.pallas_call(
        body,
        grid=(q // _BQ,),
        in_specs=[
            pl.BlockSpec((_BQ, d), lambda qi: (qi, 0)),
            pl.BlockSpec((k, d), lambda qi: (0, 0)),
            pl.BlockSpec((_BQ, 1), lambda qi: (qi, 0)),
            pl.BlockSpec((1, k), lambda qi: (0, 0)),
        ],
        out_specs=[
            pl.BlockSpec((_BQ,), lambda qi: (qi,)),
            pl.BlockSpec((_BQ,), lambda qi: (qi,)),
            pl.BlockSpec((1, 1), lambda qi: (0, 0)),
        ],
        out_shape=[
            jax.ShapeDtypeStruct((q,), jnp.float32),
            jax.ShapeDtypeStruct((q,), jnp.int32),
            jax.ShapeDtypeStruct((1, 1), jnp.float32),
        ],
        compiler_params=pltpu.CompilerParams(
            vmem_limit_bytes=60 * 1024 * 1024,
        ),
    )(f2, patch_memory, q_sq, m_sq)
    return min_d, nn_idx, score[0, 0]


# speedup vs baseline: 1.2255x; 1.2255x over previous
import functools

import jax
import jax.numpy as jnp
from jax.experimental import pallas as pl
from jax.experimental.pallas import tpu as pltpu

_BQ = 256
_BK = 4096


def _patchcore_kernel(f2_ref, m_ref, qsq_ref, msq_ref,
                      min_ref, idx_ref, score_ref, *, n_chunks, k_total):
    f2 = f2_ref[...]
    q_sq = qsq_ref[...]

    def body(ki, carry):
        best_d, best_i = carry
        m = m_ref[pl.ds(ki * _BK, _BK), :]
        m_sq = msq_ref[:, pl.ds(ki * _BK, _BK)]
        cross2 = jax.lax.dot_general(
            f2, m, (((1,), (1,)), ((), ())),
            preferred_element_type=jnp.float32)
        dist = jnp.sqrt(jnp.maximum((q_sq + m_sq) + cross2, 0.0))
        bminv = jnp.min(dist, axis=1)
        bidx = jnp.argmin(dist, axis=1).astype(jnp.int32)
        bidx = bidx + ki * _BK
        take = bminv < best_d
        return (jnp.where(take, bminv, best_d),
                jnp.where(take, bidx, best_i))

    init = (jnp.full((_BQ,), jnp.inf, dtype=jnp.float32),
            jnp.zeros((_BQ,), dtype=jnp.int32))
    best_d, best_i = jax.lax.fori_loop(0, n_chunks, body, init)
    min_ref[...] = best_d
    idx_ref[...] = best_i

    block_max = jnp.max(best_d)[None, None]
    qi = pl.program_id(0)

    @pl.when(qi == 0)
    def _():
        score_ref[...] = block_max

    @pl.when(qi != 0)
    def _():
        score_ref[...] = jnp.maximum(score_ref[...], block_max)


def kernel(features, patch_memory):
    q, d = features.shape
    k, _ = patch_memory.shape
    n_chunks = k // _BK

    q_sq = jnp.sum(features * features, axis=1, keepdims=True)
    m_sq = jnp.sum(patch_memory * patch_memory, axis=1)[None, :]
    f2 = features * -2.0

    body = functools.partial(_patchcore_kernel, n_chunks=n_chunks, k_total=k)

    min_d, nn_idx, score = pl.pallas_call(
        body,
        grid=(q // _BQ,),
        in_specs=[
            pl.BlockSpec((_BQ, d), lambda qi: (qi, 0)),
            pl.BlockSpec((k, d), lambda qi: (0, 0)),
            pl.BlockSpec((_BQ, 1), lambda qi: (qi, 0)),
            pl.BlockSpec((1, k), lambda qi: (0, 0)),
        ],
        out_specs=[
            pl.BlockSpec((_BQ,), lambda qi: (qi,)),
            pl.BlockSpec((_BQ,), lambda qi: (qi,)),
            pl.BlockSpec((1, 1), lambda qi: (0, 0)),
        ],
        out_shape=[
            jax.ShapeDtypeStruct((q,), jnp.float32),
            jax.ShapeDtypeStruct((q,), jnp.int32),
            jax.ShapeDtypeStruct((1, 1), jnp.float32),
        ],
        compiler_params=pltpu.CompilerParams(
            vmem_limit_bytes=60 * 1024 * 1024,
        ),
    )(f2, patch_memory, q_sq, m_sq)
    return min_d, nn_idx, score[0, 0]


# speedup vs baseline: 1.3300x; 1.0853x over previous
import functools

import jax
import jax.numpy as jnp
from jax.experimental import pallas as pl
from jax.experimental.pallas import tpu as pltpu

_BQ = 256
_BK = 8192


def _patchcore_kernel(f2_ref, m_ref, qsq_ref, msq_ref,
                      min_ref, idx_ref, score_ref, *, n_chunks, k_total):
    f2 = f2_ref[...]
    q_sq = qsq_ref[...]

    def body(ki, carry):
        best_d, best_i = carry
        m = m_ref[pl.ds(ki * _BK, _BK), :]
        m_sq = msq_ref[:, pl.ds(ki * _BK, _BK)]
        cross2 = jax.lax.dot_general(
            f2, m, (((1,), (1,)), ((), ())),
            preferred_element_type=jnp.float32)
        dist = jnp.sqrt(jnp.maximum((q_sq + m_sq) + cross2, 0.0))
        bminv = jnp.min(dist, axis=1)
        bidx = jnp.argmin(dist, axis=1).astype(jnp.int32)
        bidx = bidx + ki * _BK
        take = bminv < best_d
        return (jnp.where(take, bminv, best_d),
                jnp.where(take, bidx, best_i))

    init = (jnp.full((_BQ,), jnp.inf, dtype=jnp.float32),
            jnp.zeros((_BQ,), dtype=jnp.int32))
    best_d, best_i = jax.lax.fori_loop(0, n_chunks, body, init)
    min_ref[...] = best_d
    idx_ref[...] = best_i

    block_max = jnp.max(best_d)[None, None]
    qi = pl.program_id(0)

    @pl.when(qi == 0)
    def _():
        score_ref[...] = block_max

    @pl.when(qi != 0)
    def _():
        score_ref[...] = jnp.maximum(score_ref[...], block_max)


def kernel(features, patch_memory):
    q, d = features.shape
    k, _ = patch_memory.shape
    n_chunks = k // _BK

    q_sq = jnp.sum(features * features, axis=1, keepdims=True)
    m_sq = jnp.sum(patch_memory * patch_memory, axis=1)[None, :]
    f2 = features * -2.0

    body = functools.partial(_patchcore_kernel, n_chunks=n_chunks, k_total=k)

    min_d, nn_idx, score = pl.pallas_call(
        body,
        grid=(q // _BQ,),
        in_specs=[
            pl.BlockSpec((_BQ, d), lambda qi: (qi, 0)),
            pl.BlockSpec((k, d), lambda qi: (0, 0)),
            pl.BlockSpec((_BQ, 1), lambda qi: (qi, 0)),
            pl.BlockSpec((1, k), lambda qi: (0, 0)),
        ],
        out_specs=[
            pl.BlockSpec((_BQ,), lambda qi: (qi,)),
            pl.BlockSpec((_BQ,), lambda qi: (qi,)),
            pl.BlockSpec((1, 1), lambda qi: (0, 0)),
        ],
        out_shape=[
            jax.ShapeDtypeStruct((q,), jnp.float32),
            jax.ShapeDtypeStruct((q,), jnp.int32),
            jax.ShapeDtypeStruct((1, 1), jnp.float32),
        ],
        compiler_params=pltpu.CompilerParams(
            vmem_limit_bytes=60 * 1024 * 1024,
        ),
    )(f2, patch_memory, q_sq, m_sq)
    return min_d, nn_idx, score[0, 0]


# speedup vs baseline: 1.4050x; 1.0564x over previous
import functools

import jax
import jax.numpy as jnp
from jax.experimental import pallas as pl
from jax.experimental.pallas import tpu as pltpu

_BQ = 256
_BK = 16384


def _patchcore_kernel(f2_ref, m_ref, qsq_ref, msq_ref,
                      min_ref, idx_ref, score_ref, *, n_chunks, k_total):
    f2 = f2_ref[...]
    q_sq = qsq_ref[...]

    def body(ki, carry):
        best_d, best_i = carry
        m = m_ref[pl.ds(ki * _BK, _BK), :]
        m_sq = msq_ref[:, pl.ds(ki * _BK, _BK)]
        cross2 = jax.lax.dot_general(
            f2, m, (((1,), (1,)), ((), ())),
            preferred_element_type=jnp.float32)
        dist = jnp.sqrt(jnp.maximum((q_sq + m_sq) + cross2, 0.0))
        bminv = jnp.min(dist, axis=1)
        bidx = jnp.argmin(dist, axis=1).astype(jnp.int32)
        bidx = bidx + ki * _BK
        take = bminv < best_d
        return (jnp.where(take, bminv, best_d),
                jnp.where(take, bidx, best_i))

    init = (jnp.full((_BQ,), jnp.inf, dtype=jnp.float32),
            jnp.zeros((_BQ,), dtype=jnp.int32))
    best_d, best_i = jax.lax.fori_loop(0, n_chunks, body, init)
    min_ref[...] = best_d
    idx_ref[...] = best_i

    block_max = jnp.max(best_d)[None, None]
    qi = pl.program_id(0)

    @pl.when(qi == 0)
    def _():
        score_ref[...] = block_max

    @pl.when(qi != 0)
    def _():
        score_ref[...] = jnp.maximum(score_ref[...], block_max)


def kernel(features, patch_memory):
    q, d = features.shape
    k, _ = patch_memory.shape
    n_chunks = k // _BK

    q_sq = jnp.sum(features * features, axis=1, keepdims=True)
    m_sq = jnp.sum(patch_memory * patch_memory, axis=1)[None, :]
    f2 = features * -2.0

    body = functools.partial(_patchcore_kernel, n_chunks=n_chunks, k_total=k)

    min_d, nn_idx, score = pl.pallas_call(
        body,
        grid=(q // _BQ,),
        in_specs=[
            pl.BlockSpec((_BQ, d), lambda qi: (qi, 0)),
            pl.BlockSpec((k, d), lambda qi: (0, 0)),
            pl.BlockSpec((_BQ, 1), lambda qi: (qi, 0)),
            pl.BlockSpec((1, k), lambda qi: (0, 0)),
        ],
        out_specs=[
            pl.BlockSpec((_BQ,), lambda qi: (qi,)),
            pl.BlockSpec((_BQ,), lambda qi: (qi,)),
            pl.BlockSpec((1, 1), lambda qi: (0, 0)),
        ],
        out_shape=[
            jax.ShapeDtypeStruct((q,), jnp.float32),
            jax.ShapeDtypeStruct((q,), jnp.int32),
            jax.ShapeDtypeStruct((1, 1), jnp.float32),
        ],
        compiler_params=pltpu.CompilerParams(
            vmem_limit_bytes=60 * 1024 * 1024,
        ),
    )(f2, patch_memory, q_sq, m_sq)
    return min_d, nn_idx, score[0, 0]
